# baseline (speedup 1.0000x reference)
import functools

import jax
import jax.numpy as jnp
from jax import lax
from jax.experimental import pallas as pl
from jax.experimental.pallas import tpu as pltpu
from jax.experimental.pallas import tpu_sc as plsc

_NC = 2
_NS = 16
_NW = _NC * _NS


def _sc_gather_combo(table, idx):
    R = idx.shape[0]
    W = table.shape[1]
    rpw = R // _NW
    C = 128
    nch = rpw // C
    mesh = plsc.VectorSubcoreMesh(core_axis_name="c", subcore_axis_name="s")

    @functools.partial(
        pl.kernel,
        mesh=mesh,
        out_type=jax.ShapeDtypeStruct((R, W), jnp.int32),
        scratch_types=[
            pltpu.VMEM((rpw,), jnp.int32),
            pltpu.VMEM((rpw, W), jnp.int32),
            pltpu.SemaphoreType.DMA,
        ],
        compiler_params=pltpu.CompilerParams(use_tc_tiling_on_sc=False),
    )
    def k(table_hbm, idx_hbm, out_hbm, idx_v, rows_v, sem):
        wid = lax.axis_index("s") * _NC + lax.axis_index("c")
        base = wid * rpw
        pltpu.sync_copy(idx_hbm.at[pl.ds(base, rpw)], idx_v)
        cps = [
            pltpu.async_copy(
                table_hbm.at[idx_v.at[pl.ds(c * C, C)]],
                rows_v.at[pl.ds(c * C, C)],
                sem,
            )
            for c in range(nch)
        ]
        for cp in cps:
            cp.wait()
        pltpu.sync_copy(rows_v, out_hbm.at[pl.ds(base, rpw)])

    return k(table, idx)


def _sc_gather_rows(table, idx):
    R = idx.shape[0]
    D = table.shape[1]
    rpw = R // _NW
    C = 64
    NB = 4
    nch = rpw // C
    ngrp = nch // NB
    mesh = plsc.VectorSubcoreMesh(core_axis_name="c", subcore_axis_name="s")

    @functools.partial(
        pl.kernel,
        mesh=mesh,
        out_type=jax.ShapeDtypeStruct((R, D), jnp.float32),
        scratch_types=[
            pltpu.VMEM((rpw,), jnp.int32),
            pltpu.VMEM((NB, C, D), jnp.float32),
            pltpu.SemaphoreType.DMA,
            pltpu.SemaphoreType.DMA,
            pltpu.SemaphoreType.DMA,
            pltpu.SemaphoreType.DMA,
            pltpu.SemaphoreType.DMA,
            pltpu.SemaphoreType.DMA,
            pltpu.SemaphoreType.DMA,
            pltpu.SemaphoreType.DMA,
        ],
    )
    def k(table_hbm, idx_hbm, out_hbm, idx_v, buf,
          g0, g1, g2, g3, s0, s1, s2, s3):
        gsem = (g0, g1, g2, g3)
        ssem = (s0, s1, s2, s3)
        wid = lax.axis_index("s") * _NC + lax.axis_index("c")
        base = wid * rpw
        pltpu.sync_copy(idx_hbm.at[pl.ds(base, rpw)], idx_v)

        def start_gather(ch, b):
            pltpu.async_copy(
                table_hbm.at[idx_v.at[pl.ds(ch * C, C)]], buf.at[b], gsem[b])

        def wait_gather(b):
            pltpu.make_async_copy(
                table_hbm.at[idx_v.at[pl.ds(0, C)]], buf.at[b], gsem[b]
            ).wait()

        def start_store(ch, b):
            pltpu.async_copy(
                buf.at[b], out_hbm.at[pl.ds(base + ch * C, C)], ssem[b])

        def wait_store(b):
            pltpu.make_async_copy(
                buf.at[b], out_hbm.at[pl.ds(base, C)], ssem[b]
            ).wait()

        for b in range(NB):
            start_gather(b, b)

        def body(g, carry):
            ch0 = g * NB
            for b in range(NB):
                wait_gather(b)
                start_store(ch0 + b, b)
            for b in range(NB):
                wait_store(b)

                @pl.when(g < ngrp - 1)
                def _(b=b):
                    start_gather(ch0 + NB + b, b)

            return carry

        lax.fori_loop(0, ngrp, body, 0)

    return k(table, idx)


def _dense_body(L, S, D,
                mask_ref, wcol_ref, ev0_ref, item_ref, ev1_ref,
                w1a_ref, r_ref, w2_ref, w3_ref, out_ref):
    LS = L * S
    mask = mask_ref[0]
    item_e = item_ref[0]
    ev0 = ev0_ref[0]
    ev1 = ev1_ref[0]
    wcol = wcol_ref[0]

    denom_items = jnp.sum(mask)
    sess = jnp.dot(mask, item_e, preferred_element_type=jnp.float32)
    sess = sess / denom_items

    m = sess * ev1
    pre = jnp.dot(m, w1a_ref[...], preferred_element_type=jnp.float32)
    pre = pre + wcol * r_ref[...]
    pre = jnp.where(pre >= 0.0, pre, 0.2 * pre)
    logit = jnp.dot(pre, w2_ref[...], preferred_element_type=jnp.float32)
    e = jnp.exp(logit)

    row = lax.broadcasted_iota(jnp.int32, (L, LS), 0)
    col = lax.broadcasted_iota(jnp.int32, (L, LS), 1)
    off = col - row * S
    msel = jnp.where((off >= 0) & (off < S), 1.0, 0.0)
    rowt = lax.broadcasted_iota(jnp.int32, (LS, L), 0)
    colt = lax.broadcasted_iota(jnp.int32, (LS, L), 1)
    offt = rowt - colt * S
    mselt = jnp.where((offt >= 0) & (offt < S), 1.0, 0.0)

    denom = jnp.dot(msel, e, preferred_element_type=jnp.float32)
    dback = jnp.dot(mselt, denom, preferred_element_type=jnp.float32)
    alpha = e / dback
    nv = jnp.dot(msel, alpha * ev1, preferred_element_type=jnp.float32)

    w3a = w3_ref[0:D, :]
    w3b = w3_ref[D:2 * D, :]
    out = jnp.dot(ev0, w3a, preferred_element_type=jnp.float32)
    out = out + jnp.dot(nv, w3b, preferred_element_type=jnp.float32)
    out_ref[0] = jnp.maximum(out, 0.0)


def _tc_dense(mask3, wcol3, ev0_3, item3, ev1_3, w1a, r_row, w2, w3):
    B, L, D = ev0_3.shape
    LS = ev1_3.shape[1]
    S = LS // L
    body = functools.partial(_dense_body, L, S, D)
    return pl.pallas_call(
        body,
        grid=(B,),
        in_specs=[
            pl.BlockSpec((1, 1, L), lambda b: (b, 0, 0)),
            pl.BlockSpec((1, LS, 1), lambda b: (b, 0, 0)),
            pl.BlockSpec((1, L, D), lambda b: (b, 0, 0)),
            pl.BlockSpec((1, L, D), lambda b: (b, 0, 0)),
            pl.BlockSpec((1, LS, D), lambda b: (b, 0, 0)),
            pl.BlockSpec((D, D), lambda b: (0, 0)),
            pl.BlockSpec((1, D), lambda b: (0, 0)),
            pl.BlockSpec((D, 1), lambda b: (0, 0)),
            pl.BlockSpec((2 * D, D), lambda b: (0, 0)),
        ],
        out_specs=pl.BlockSpec((1, L, D), lambda b: (b, 0, 0)),
        out_shape=jax.ShapeDtypeStruct((B, L, D), jnp.float32),
    )(mask3, wcol3, ev0_3, item3, ev1_3, w1a, r_row, w2, w3)


def kernel(inputs, mask_item, item, adj_all, num_w, embedding, w1, w2, w3):
    B, L = inputs.shape
    N, S = adj_all.shape
    D = embedding.shape[1]
    BL = B * L

    flat = inputs.reshape(-1).astype(jnp.int32)
    combo = jnp.concatenate(
        [
            adj_all.astype(jnp.int32),
            lax.bitcast_convert_type(num_w.astype(jnp.float32), jnp.int32),
            jnp.zeros((N, 32 - 2 * S), jnp.int32),
        ],
        axis=1,
    )
    rows = _sc_gather_combo(combo, flat)
    neigh = rows[:, :S].reshape(-1)
    weight = lax.bitcast_convert_type(rows[:, S:2 * S], jnp.float32)

    idx_all = jnp.concatenate(
        [flat, item.reshape(-1).astype(jnp.int32), neigh])
    g = _sc_gather_rows(embedding.astype(jnp.float32), idx_all)
    ev0_3 = g[:BL].reshape(B, L, D)
    item3 = g[BL:2 * BL].reshape(B, L, D)
    ev1_3 = g[2 * BL:].reshape(B, L * S, D)

    mask3 = mask_item.astype(jnp.float32).reshape(B, 1, L)
    wcol3 = weight.reshape(B, L * S, 1)
    w1a = w1[:D, :]
    r_row = w1[D:D + 1, :]
    return _tc_dense(mask3, wcol3, ev0_3, item3, ev1_3, w1a, r_row, w2, w3)


# speedup vs baseline: 2.3403x; 2.3403x over previous
import functools

import jax
import jax.numpy as jnp
from jax import lax
from jax.experimental import pallas as pl
from jax.experimental.pallas import tpu as pltpu
from jax.experimental.pallas import tpu_sc as plsc

_NC = 2
_NS = 16
_NW = _NC * _NS

_G = 8


def _sc_gather_combo(table, idx):
    R = idx.shape[0]
    W = table.shape[1]
    rpw = R // _NW
    C = 128
    nch = rpw // C
    mesh = plsc.VectorSubcoreMesh(core_axis_name="c", subcore_axis_name="s")

    @functools.partial(
        pl.kernel,
        mesh=mesh,
        out_type=jax.ShapeDtypeStruct((R, W), jnp.int32),
        scratch_types=[
            pltpu.VMEM((rpw,), jnp.int32),
            pltpu.VMEM((rpw, W), jnp.int32),
            pltpu.SemaphoreType.DMA,
        ],
        compiler_params=pltpu.CompilerParams(use_tc_tiling_on_sc=False),
    )
    def k(table_hbm, idx_hbm, out_hbm, idx_v, rows_v, sem):
        wid = lax.axis_index("s") * _NC + lax.axis_index("c")
        base = wid * rpw
        pltpu.sync_copy(idx_hbm.at[pl.ds(base, rpw)], idx_v)
        cps = [
            pltpu.async_copy(
                table_hbm.at[idx_v.at[pl.ds(c * C, C)]],
                rows_v.at[pl.ds(c * C, C)],
                sem,
            )
            for c in range(nch)
        ]
        for cp in cps:
            cp.wait()
        pltpu.sync_copy(rows_v, out_hbm.at[pl.ds(base, rpw)])

    return k(table, idx)


def _sc_gather_rows(table, idx):
    R = idx.shape[0]
    D = table.shape[1]
    rpw = R // _NW
    C = 64
    NB = 4
    nch = rpw // C
    ngrp = nch // NB
    mesh = plsc.VectorSubcoreMesh(core_axis_name="c", subcore_axis_name="s")

    @functools.partial(
        pl.kernel,
        mesh=mesh,
        out_type=jax.ShapeDtypeStruct((R, D), jnp.float32),
        scratch_types=[
            pltpu.VMEM((rpw,), jnp.int32),
            pltpu.VMEM((NB, C, D), jnp.float32),
            pltpu.SemaphoreType.DMA,
            pltpu.SemaphoreType.DMA,
            pltpu.SemaphoreType.DMA,
            pltpu.SemaphoreType.DMA,
            pltpu.SemaphoreType.DMA,
            pltpu.SemaphoreType.DMA,
            pltpu.SemaphoreType.DMA,
            pltpu.SemaphoreType.DMA,
        ],
    )
    def k(table_hbm, idx_hbm, out_hbm, idx_v, buf,
          g0, g1, g2, g3, s0, s1, s2, s3):
        gsem = (g0, g1, g2, g3)
        ssem = (s0, s1, s2, s3)
        wid = lax.axis_index("s") * _NC + lax.axis_index("c")
        base = wid * rpw
        pltpu.sync_copy(idx_hbm.at[pl.ds(base, rpw)], idx_v)

        def start_gather(ch, b):
            pltpu.async_copy(
                table_hbm.at[idx_v.at[pl.ds(ch * C, C)]], buf.at[b], gsem[b])

        def wait_gather(b):
            pltpu.make_async_copy(
                table_hbm.at[idx_v.at[pl.ds(0, C)]], buf.at[b], gsem[b]
            ).wait()

        def start_store(ch, b):
            pltpu.async_copy(
                buf.at[b], out_hbm.at[pl.ds(base + ch * C, C)], ssem[b])

        def wait_store(b):
            pltpu.make_async_copy(
                buf.at[b], out_hbm.at[pl.ds(base, C)], ssem[b]
            ).wait()

        for b in range(NB):
            start_gather(b, b)

        def body(g, carry):
            ch0 = g * NB
            for b in range(NB):
                wait_gather(b)
                start_store(ch0 + b, b)
            for b in range(NB):
                wait_store(b)

                @pl.when(g < ngrp - 1)
                def _(b=b):
                    start_gather(ch0 + NB + b, b)

            return carry

        lax.fori_loop(0, ngrp, body, 0)

    return k(table, idx)


def _dense_body(L, S, D,
                gall_ref, wcol_ref, maskc_ref,
                msum_ref, mexp_ref, msel_ref, mselt_ref, mpair_ref,
                w1a_ref, r_ref, w2_ref, w3_ref, out_ref, nvbuf):
    G = _G
    GL = G * L
    GLS = G * L * S
    bf = jnp.bfloat16

    ev0 = gall_ref[0, pl.ds(0, GL), :]
    item_e = gall_ref[0, pl.ds(GL, GL), :]
    ev1 = gall_ref[0, pl.ds(2 * GL, GLS), :]
    wcol = wcol_ref[0]
    maskc = maskc_ref[0]

    masked = item_e * maskc
    sess = jnp.dot(msum_ref[...], masked, preferred_element_type=jnp.float32)
    dn = jnp.dot(msum_ref[...], maskc, preferred_element_type=jnp.float32)
    sess = sess / dn
    sess_exp = jnp.dot(mexp_ref[...], sess.astype(bf),
                       preferred_element_type=jnp.float32)

    m = sess_exp * ev1
    pre = jnp.dot(m.astype(bf), w1a_ref[...],
                  preferred_element_type=jnp.float32)
    pre = pre + wcol * r_ref[...]
    pre = jnp.where(pre >= 0.0, pre, 0.2 * pre)
    logit = jnp.dot(pre, w2_ref[...], preferred_element_type=jnp.float32)
    e = jnp.exp(logit)

    den = jnp.dot(msel_ref[...], e, preferred_element_type=jnp.float32)
    dback = jnp.dot(mselt_ref[...], den, preferred_element_type=jnp.float32)
    alpha = e / dback
    weighted = (alpha * ev1).astype(bf)

    npair = G // 2
    PLS = 2 * L * S
    PL = 2 * L
    for p in range(npair):
        wsl = lax.slice(weighted, (p * PLS, 0), ((p + 1) * PLS, D))
        nvbuf[pl.ds(p * PL, PL), :] = jnp.dot(
            mpair_ref[...], wsl, preferred_element_type=jnp.float32)

    nv = nvbuf[...]
    w3a = w3_ref[pl.ds(0, D), :]
    w3b = w3_ref[pl.ds(D, D), :]
    out = jnp.dot(ev0.astype(bf), w3a, preferred_element_type=jnp.float32)
    out = out + jnp.dot(nv.astype(bf), w3b, preferred_element_type=jnp.float32)
    out_ref[0] = jnp.maximum(out, 0.0)


def _tc_dense(gall, wcol3, maskc3, w1, w2, w3, B, L, S, D):
    G = _G
    NG = B // G
    GL = G * L
    GLS = G * L * S

    msum = (jnp.arange(GL)[None, :] // L ==
            jnp.arange(G)[:, None]).astype(jnp.float32)
    mexp = (jnp.arange(GLS)[:, None] // (L * S) ==
            jnp.arange(G)[None, :]).astype(jnp.bfloat16)
    msel = (jnp.arange(GLS)[None, :] // S ==
            jnp.arange(GL)[:, None]).astype(jnp.float32)
    mselt = (jnp.arange(GLS)[:, None] // S ==
             jnp.arange(GL)[None, :]).astype(jnp.float32)
    mpair = (jnp.arange(2 * L * S)[None, :] // S ==
             jnp.arange(2 * L)[:, None]).astype(jnp.bfloat16)

    w1a = w1[:D, :].astype(jnp.bfloat16)
    r_row = w1[D:D + 1, :]
    w2f = w2.astype(jnp.float32)
    w3b16 = w3.astype(jnp.bfloat16)

    body = functools.partial(_dense_body, L, S, D)
    out = pl.pallas_call(
        body,
        grid=(NG,),
        in_specs=[
            pl.BlockSpec((1, 2 * GL + GLS, D), lambda b: (b, 0, 0)),
            pl.BlockSpec((1, GLS, 1), lambda b: (b, 0, 0)),
            pl.BlockSpec((1, GL, 1), lambda b: (b, 0, 0)),
            pl.BlockSpec((G, GL), lambda b: (0, 0)),
            pl.BlockSpec((GLS, G), lambda b: (0, 0)),
            pl.BlockSpec((GL, GLS), lambda b: (0, 0)),
            pl.BlockSpec((GLS, GL), lambda b: (0, 0)),
            pl.BlockSpec((2 * L, 2 * L * S), lambda b: (0, 0)),
            pl.BlockSpec((D, D), lambda b: (0, 0)),
            pl.BlockSpec((1, D), lambda b: (0, 0)),
            pl.BlockSpec((D, 1), lambda b: (0, 0)),
            pl.BlockSpec((2 * D, D), lambda b: (0, 0)),
        ],
        out_specs=pl.BlockSpec((1, GL, D), lambda b: (b, 0, 0)),
        out_shape=jax.ShapeDtypeStruct((NG, GL, D), jnp.float32),
        scratch_shapes=[pltpu.VMEM((GL, D), jnp.float32)],
    )(gall, wcol3, maskc3, msum, mexp, msel, mselt, mpair,
      w1a, r_row, w2f, w3b16)
    return out.reshape(B, L, D)


def kernel(inputs, mask_item, item, adj_all, num_w, embedding, w1, w2, w3):
    B, L = inputs.shape
    N, S = adj_all.shape
    D = embedding.shape[1]
    G = _G
    NG = B // G

    flat = inputs.reshape(-1).astype(jnp.int32)
    combo = jnp.concatenate(
        [
            adj_all.astype(jnp.int32),
            lax.bitcast_convert_type(num_w.astype(jnp.float32), jnp.int32),
            jnp.zeros((N, 32 - 2 * S), jnp.int32),
        ],
        axis=1,
    )
    rows = _sc_gather_combo(combo, flat)
    neigh = rows[:, :S].reshape(-1).astype(jnp.int32)
    weight = lax.bitcast_convert_type(rows[:, S:2 * S], jnp.float32)

    idx_all = jnp.concatenate(
        [
            flat.reshape(NG, G * L),
            item.reshape(NG, G * L).astype(jnp.int32),
            neigh.reshape(NG, G * L * S),
        ],
        axis=1,
    ).reshape(-1)
    g = _sc_gather_rows(embedding.astype(jnp.float32), idx_all)
    gall = g.reshape(NG, (2 + S) * G * L, D)

    wcol3 = weight.reshape(NG, G * L * S, 1)
    maskc3 = mask_item.astype(jnp.float32).reshape(NG, G * L, 1)
    return _tc_dense(gall, wcol3, maskc3, w1, w2, w3, B, L, S, D)


# speedup vs baseline: 2.8125x; 1.2018x over previous
import functools

import jax
import jax.numpy as jnp
from jax import lax
from jax.experimental import pallas as pl
from jax.experimental.pallas import tpu as pltpu
from jax.experimental.pallas import tpu_sc as plsc

_NC = 2
_NS = 16
_NW = _NC * _NS

_G = 16


def _sc_gather_combo(table, idx):
    R = idx.shape[0]
    W = table.shape[1]
    rpw = R // _NW
    C = 128
    nch = rpw // C
    mesh = plsc.VectorSubcoreMesh(core_axis_name="c", subcore_axis_name="s")

    @functools.partial(
        pl.kernel,
        mesh=mesh,
        out_type=jax.ShapeDtypeStruct((R, W), jnp.int32),
        scratch_types=[
            pltpu.VMEM((rpw,), jnp.int32),
            pltpu.VMEM((rpw, W), jnp.int32),
            pltpu.SemaphoreType.DMA,
        ],
        compiler_params=pltpu.CompilerParams(use_tc_tiling_on_sc=False),
    )
    def k(table_hbm, idx_hbm, out_hbm, idx_v, rows_v, sem):
        wid = lax.axis_index("s") * _NC + lax.axis_index("c")
        base = wid * rpw
        pltpu.sync_copy(idx_hbm.at[pl.ds(base, rpw)], idx_v)
        cps = [
            pltpu.async_copy(
                table_hbm.at[idx_v.at[pl.ds(c * C, C)]],
                rows_v.at[pl.ds(c * C, C)],
                sem,
            )
            for c in range(nch)
        ]
        for cp in cps:
            cp.wait()
        pltpu.sync_copy(rows_v, out_hbm.at[pl.ds(base, rpw)])

    return k(table, idx)


def _sc_gather_rows(table, idx):
    R = idx.shape[0]
    D = table.shape[1]
    rpw = R // _NW
    C = 64
    NB = 4
    nch = rpw // C
    ngrp = nch // NB
    mesh = plsc.VectorSubcoreMesh(core_axis_name="c", subcore_axis_name="s")

    @functools.partial(
        pl.kernel,
        mesh=mesh,
        out_type=jax.ShapeDtypeStruct((R, D), jnp.float32),
        scratch_types=[
            pltpu.VMEM((rpw,), jnp.int32),
            pltpu.VMEM((NB, C, D), jnp.float32),
            pltpu.SemaphoreType.DMA,
            pltpu.SemaphoreType.DMA,
            pltpu.SemaphoreType.DMA,
            pltpu.SemaphoreType.DMA,
            pltpu.SemaphoreType.DMA,
            pltpu.SemaphoreType.DMA,
            pltpu.SemaphoreType.DMA,
            pltpu.SemaphoreType.DMA,
        ],
    )
    def k(table_hbm, idx_hbm, out_hbm, idx_v, buf,
          g0, g1, g2, g3, s0, s1, s2, s3):
        gsem = (g0, g1, g2, g3)
        ssem = (s0, s1, s2, s3)
        wid = lax.axis_index("s") * _NC + lax.axis_index("c")
        base = wid * rpw
        pltpu.sync_copy(idx_hbm.at[pl.ds(base, rpw)], idx_v)

        def start_gather(ch, b):
            pltpu.async_copy(
                table_hbm.at[idx_v.at[pl.ds(ch * C, C)]], buf.at[b], gsem[b])

        def wait_gather(b):
            pltpu.make_async_copy(
                table_hbm.at[idx_v.at[pl.ds(0, C)]], buf.at[b], gsem[b]
            ).wait()

        def start_store(ch, b):
            pltpu.async_copy(
                buf.at[b], out_hbm.at[pl.ds(base + ch * C, C)], ssem[b])

        def wait_store(b):
            pltpu.make_async_copy(
                buf.at[b], out_hbm.at[pl.ds(base, C)], ssem[b]
            ).wait()

        for b in range(NB):
            start_gather(b, b)

        def body(g, carry):
            ch0 = g * NB
            for b in range(NB):
                wait_gather(b)
                start_store(ch0 + b, b)
            for b in range(NB):
                wait_store(b)

                @pl.when(g < ngrp - 1)
                def _(b=b):
                    start_gather(ch0 + NB + b, b)

            return carry

        lax.fori_loop(0, ngrp, body, 0)

    return k(table, idx)


def _dense_body(L, S, D,
                gall_ref, wcol_ref, maskc_ref,
                msum_ref, mexp_ref, msel_ref, mpair_ref,
                w1a_ref, r_ref, w2_ref, w3_ref, out_ref, nvbuf):
    G = _G
    GL = G * L
    GLS = G * L * S
    bf = jnp.bfloat16

    ev0 = gall_ref[0, pl.ds(0, GL), :]
    item_e = gall_ref[0, pl.ds(GL, GL), :]
    ev1 = gall_ref[0, pl.ds(2 * GL, GLS), :]
    wcol = wcol_ref[0]
    maskc = maskc_ref[0]

    masked = item_e * maskc
    sess = jnp.dot(msum_ref[...], masked, preferred_element_type=jnp.float32)
    dn = jnp.dot(msum_ref[...], maskc, preferred_element_type=jnp.float32)
    sess = sess / dn
    sess_exp = jnp.dot(mexp_ref[...], sess.astype(bf),
                       preferred_element_type=jnp.float32)

    m = sess_exp * ev1
    pre = jnp.dot(m.astype(bf), w1a_ref[...],
                  preferred_element_type=jnp.float32)
    pre = pre + wcol * r_ref[...]
    pre = jnp.where(pre >= 0.0, pre, 0.2 * pre)
    logit = jnp.dot(pre.astype(bf), w2_ref[...],
                    preferred_element_type=jnp.float32)
    e = jnp.exp(logit)
    e_bf = e.astype(bf)

    den = jnp.dot(msel_ref[...], e_bf, preferred_element_type=jnp.float32)
    weighted = e_bf * ev1.astype(bf)

    npair = G // 2
    PLS = 2 * L * S
    PL = 2 * L
    for p in range(npair):
        wsl = lax.slice(weighted, (p * PLS, 0), ((p + 1) * PLS, D))
        nvbuf[pl.ds(p * PL, PL), :] = jnp.dot(
            mpair_ref[...], wsl, preferred_element_type=jnp.float32)

    nv = nvbuf[...] / den
    w3a = w3_ref[pl.ds(0, D), :]
    w3b = w3_ref[pl.ds(D, D), :]
    out = jnp.dot(ev0.astype(bf), w3a, preferred_element_type=jnp.float32)
    out = out + jnp.dot(nv.astype(bf), w3b, preferred_element_type=jnp.float32)
    out_ref[0] = jnp.maximum(out, 0.0)


def _tc_dense(gall, wcol3, maskc3, w1, w2, w3, B, L, S, D):
    G = _G
    NG = B // G
    GL = G * L
    GLS = G * L * S

    msum = (jnp.arange(GL)[None, :] // L ==
            jnp.arange(G)[:, None]).astype(jnp.float32)
    mexp = (jnp.arange(GLS)[:, None] // (L * S) ==
            jnp.arange(G)[None, :]).astype(jnp.bfloat16)
    msel = (jnp.arange(GLS)[None, :] // S ==
            jnp.arange(GL)[:, None]).astype(jnp.bfloat16)
    mpair = (jnp.arange(2 * L * S)[None, :] // S ==
             jnp.arange(2 * L)[:, None]).astype(jnp.bfloat16)

    w1a = w1[:D, :].astype(jnp.bfloat16)
    r_row = w1[D:D + 1, :]
    w2f = w2.astype(jnp.bfloat16)
    w3b16 = w3.astype(jnp.bfloat16)

    body = functools.partial(_dense_body, L, S, D)
    out = pl.pallas_call(
        body,
        grid=(NG,),
        in_specs=[
            pl.BlockSpec((1, 2 * GL + GLS, D), lambda b: (b, 0, 0)),
            pl.BlockSpec((1, GLS, 1), lambda b: (b, 0, 0)),
            pl.BlockSpec((1, GL, 1), lambda b: (b, 0, 0)),
            pl.BlockSpec((G, GL), lambda b: (0, 0)),
            pl.BlockSpec((GLS, G), lambda b: (0, 0)),
            pl.BlockSpec((GL, GLS), lambda b: (0, 0)),
            pl.BlockSpec((2 * L, 2 * L * S), lambda b: (0, 0)),
            pl.BlockSpec((D, D), lambda b: (0, 0)),
            pl.BlockSpec((1, D), lambda b: (0, 0)),
            pl.BlockSpec((D, 1), lambda b: (0, 0)),
            pl.BlockSpec((2 * D, D), lambda b: (0, 0)),
        ],
        out_specs=pl.BlockSpec((1, GL, D), lambda b: (b, 0, 0)),
        out_shape=jax.ShapeDtypeStruct((NG, GL, D), jnp.float32),
        scratch_shapes=[pltpu.VMEM((GL, D), jnp.float32)],
    )(gall, wcol3, maskc3, msum, mexp, msel, mpair,
      w1a, r_row, w2f, w3b16)
    return out.reshape(B, L, D)


def kernel(inputs, mask_item, item, adj_all, num_w, embedding, w1, w2, w3):
    B, L = inputs.shape
    N, S = adj_all.shape
    D = embedding.shape[1]
    G = _G
    NG = B // G

    flat = inputs.reshape(-1).astype(jnp.int32)
    combo = jnp.concatenate(
        [
            adj_all.astype(jnp.int32),
            lax.bitcast_convert_type(num_w.astype(jnp.float32), jnp.int32),
            jnp.zeros((N, 32 - 2 * S), jnp.int32),
        ],
        axis=1,
    )
    rows = _sc_gather_combo(combo, flat)
    neigh = rows[:, :S].reshape(-1).astype(jnp.int32)
    weight = lax.bitcast_convert_type(rows[:, S:2 * S], jnp.float32)

    idx_all = jnp.concatenate(
        [
            flat.reshape(NG, G * L),
            item.reshape(NG, G * L).astype(jnp.int32),
            neigh.reshape(NG, G * L * S),
        ],
        axis=1,
    ).reshape(-1)
    g = _sc_gather_rows(embedding.astype(jnp.float32), idx_all)
    gall = g.reshape(NG, (2 + S) * G * L, D)

    wcol3 = weight.reshape(NG, G * L * S, 1)
    maskc3 = mask_item.astype(jnp.float32).reshape(NG, G * L, 1)
    return _tc_dense(gall, wcol3, maskc3, w1, w2, w3, B, L, S, D)
